# baseline (speedup 1.0000x reference)
import jax
import jax.numpy as jnp
from jax.experimental import pallas as pl


def _som_dist_kernel(x_ref, w_ref, o_ref):
    x = x_ref[...]
    w = w_ref[...]
    xm2 = x * -2.0
    xw = jax.lax.dot_general(
        xm2, w,
        dimension_numbers=(((1,), (1,)), ((), ())),
        preferred_element_type=jnp.float32,
    )
    x2 = jnp.sum(x * x, axis=1, keepdims=True)
    w2 = jnp.sum(w * w, axis=1, keepdims=True).T
    o_ref[...] = (x2 + w2) + xw


def kernel(x, weights):
    B, D = x.shape
    R, C, _ = weights.shape
    w = weights.reshape(R * C, D)
    BB = 128
    out = pl.pallas_call(
        _som_dist_kernel,
        grid=(B // BB,),
        in_specs=[
            pl.BlockSpec((BB, D), lambda i: (i, 0)),
            pl.BlockSpec((R * C, D), lambda i: (0, 0)),
        ],
        out_specs=pl.BlockSpec((BB, R * C), lambda i: (i, 0)),
        out_shape=jax.ShapeDtypeStruct((B, R * C), jnp.float32),
    )(x, w)
    return out.reshape(B, R, C)


# speedup vs baseline: 1.1954x; 1.1954x over previous
import jax
import jax.numpy as jnp
from jax.experimental import pallas as pl


def _som_dist_kernel(x_ref, w_ref, o_ref):
    x = x_ref[...]
    w = w_ref[...]
    xm2 = x * -2.0
    xw = jax.lax.dot_general(
        xm2, w,
        dimension_numbers=(((1,), (1,)), ((), ())),
        preferred_element_type=jnp.float32,
    )
    x2 = jnp.sum(x * x, axis=1, keepdims=True)
    w2 = jnp.sum(w * w, axis=1, keepdims=True).T
    o_ref[...] = (x2 + w2) + xw


def kernel(x, weights):
    B, D = x.shape
    R, C, _ = weights.shape
    w = weights.reshape(R * C, D)
    BB = 256
    out = pl.pallas_call(
        _som_dist_kernel,
        grid=(B // BB,),
        in_specs=[
            pl.BlockSpec((BB, D), lambda i: (i, 0)),
            pl.BlockSpec((R * C, D), lambda i: (0, 0)),
        ],
        out_specs=pl.BlockSpec((BB, R * C), lambda i: (i, 0)),
        out_shape=jax.ShapeDtypeStruct((B, R * C), jnp.float32),
    )(x, w)
    return out.reshape(B, R, C)
